# baseline (speedup 1.0000x reference)
import jax
import jax.numpy as jnp
from jax import lax
from jax.experimental import pallas as pl


def _fused_kernel(adj_ref, alpha_ref, uadj_ref, uops_ref, mask_ref, m_ref,
                  adj_out_ref, ops_out_ref):
    a = adj_ref[...]
    t = -jnp.log(uadj_ref[...])
    act = 1.0 / (1.0 + t * jnp.exp(-a))
    adj_out_ref[...] = act * mask_ref[...]

    e = jnp.exp(alpha_ref[...]) / (-jnp.log(uops_ref[...]))
    s = jnp.dot(e, m_ref[...], precision=lax.Precision.HIGHEST,
                preferred_element_type=jnp.float32)
    ops_out_ref[...] = e / s


def kernel(adj_para, ops_alpha, u_adj, u_ops):
    B, N, _ = adj_para.shape
    OPS = ops_alpha.shape[-1]
    LO = N * OPS
    BB = 64

    ops_alpha2 = ops_alpha.reshape(B, LO)
    u_ops2 = u_ops.reshape(B, LO)

    i = lax.broadcasted_iota(jnp.int32, (N, N), 0)
    j = lax.broadcasted_iota(jnp.int32, (N, N), 1)
    mask = (j > i).astype(jnp.float32).reshape(1, N, N)
    g = lax.iota(jnp.int32, LO) // OPS
    m = (g[:, None] == g[None, :]).astype(jnp.float32)

    grid = (B // BB,)
    adj_spec = pl.BlockSpec((BB, N, N), lambda b: (b, 0, 0))
    ops_spec = pl.BlockSpec((BB, LO), lambda b: (b, 0))
    mask_spec = pl.BlockSpec((1, N, N), lambda b: (0, 0, 0))
    m_spec = pl.BlockSpec((LO, LO), lambda b: (0, 0))

    adj_out, ops_out = pl.pallas_call(
        _fused_kernel,
        grid=grid,
        in_specs=[adj_spec, ops_spec, adj_spec, ops_spec, mask_spec, m_spec],
        out_specs=[adj_spec, ops_spec],
        out_shape=[
            jax.ShapeDtypeStruct((B, N, N), adj_para.dtype),
            jax.ShapeDtypeStruct((B, LO), ops_alpha.dtype),
        ],
    )(adj_para, ops_alpha2, u_adj, u_ops2, mask, m)

    return adj_out, ops_out.reshape(B, N, OPS)


# speedup vs baseline: 6.1321x; 6.1321x over previous
import jax
import jax.numpy as jnp
from jax import lax
from jax.experimental import pallas as pl

_TI = 8


def _adj_kernel(adj_ref, uadj_ref, out_ref):
    ti = pl.program_id(0)
    tj = pl.program_id(1)
    a = adj_ref[...]
    t = -jnp.log(uadj_ref[...])
    act = 1.0 / (1.0 + t * jnp.exp(-a))
    i = ti * _TI + lax.broadcasted_iota(jnp.int32, a.shape, 0)
    j = tj * _TI + lax.broadcasted_iota(jnp.int32, a.shape, 1)
    out_ref[...] = jnp.where(j > i, act, 0.0)


def _ops_kernel(alpha_ref, uops_ref, out_ref):
    e = jnp.exp(alpha_ref[...]) / (-jnp.log(uops_ref[...]))
    out_ref[...] = e / jnp.sum(e, axis=1, keepdims=True)


def kernel(adj_para, ops_alpha, u_adj, u_ops):
    B, N, _ = adj_para.shape
    OPS = ops_alpha.shape[-1]

    adj_t = jnp.transpose(adj_para, (1, 2, 0))
    uadj_t = jnp.transpose(u_adj, (1, 2, 0))
    alpha_t = jnp.transpose(ops_alpha, (1, 2, 0))
    uops_t = jnp.transpose(u_ops, (1, 2, 0))

    nt = N // _TI
    in_spec = pl.BlockSpec((_TI, _TI, B), lambda ti, tj: (ti, jnp.maximum(tj, ti), 0))
    out_spec = pl.BlockSpec((_TI, _TI, B), lambda ti, tj: (ti, tj, 0))
    adj_out_t = pl.pallas_call(
        _adj_kernel,
        grid=(nt, nt),
        in_specs=[in_spec, in_spec],
        out_specs=out_spec,
        out_shape=jax.ShapeDtypeStruct((N, N, B), adj_para.dtype),
    )(adj_t, uadj_t)

    IT = 8
    ops_spec = pl.BlockSpec((IT, OPS, B), lambda i: (i, 0, 0))
    ops_out_t = pl.pallas_call(
        _ops_kernel,
        grid=(N // IT,),
        in_specs=[ops_spec, ops_spec],
        out_specs=ops_spec,
        out_shape=jax.ShapeDtypeStruct((N, OPS, B), ops_alpha.dtype),
    )(alpha_t, uops_t)

    return (jnp.transpose(adj_out_t, (2, 0, 1)),
            jnp.transpose(ops_out_t, (2, 0, 1)))


# speedup vs baseline: 7.6995x; 1.2556x over previous
import jax
import jax.numpy as jnp
from jax import lax
from jax.experimental import pallas as pl

_TI = 16


def _adj_kernel(adj_ref, uadj_ref, out_ref):
    ti = pl.program_id(0)
    tj = pl.program_id(1)
    a = adj_ref[...]
    t = -jnp.log(uadj_ref[...])
    act = 1.0 / (1.0 + t * jnp.exp(-a))
    i = ti * _TI + lax.broadcasted_iota(jnp.int32, a.shape, 0)
    j = tj * _TI + lax.broadcasted_iota(jnp.int32, a.shape, 1)
    out_ref[...] = jnp.where(j > i, act, 0.0)


def _ops_kernel(alpha_ref, uops_ref, out_ref):
    e = jnp.exp(alpha_ref[...]) / (-jnp.log(uops_ref[...]))
    out_ref[...] = e / jnp.sum(e, axis=1, keepdims=True)


def kernel(adj_para, ops_alpha, u_adj, u_ops):
    B, N, _ = adj_para.shape
    OPS = ops_alpha.shape[-1]

    adj_t = jnp.transpose(adj_para, (1, 2, 0))
    uadj_t = jnp.transpose(u_adj, (1, 2, 0))
    alpha_t = jnp.transpose(ops_alpha, (1, 2, 0))
    uops_t = jnp.transpose(u_ops, (1, 2, 0))

    nt = N // _TI
    in_spec = pl.BlockSpec((_TI, _TI, B), lambda ti, tj: (ti, jnp.maximum(tj, ti), 0))
    out_spec = pl.BlockSpec((_TI, _TI, B), lambda ti, tj: (ti, tj, 0))
    adj_out_t = pl.pallas_call(
        _adj_kernel,
        grid=(nt, nt),
        in_specs=[in_spec, in_spec],
        out_specs=out_spec,
        out_shape=jax.ShapeDtypeStruct((N, N, B), adj_para.dtype),
    )(adj_t, uadj_t)

    IT = 16
    ops_spec = pl.BlockSpec((IT, OPS, B), lambda i: (i, 0, 0))
    ops_out_t = pl.pallas_call(
        _ops_kernel,
        grid=(N // IT,),
        in_specs=[ops_spec, ops_spec],
        out_specs=ops_spec,
        out_shape=jax.ShapeDtypeStruct((N, OPS, B), ops_alpha.dtype),
    )(alpha_t, uops_t)

    return (jnp.transpose(adj_out_t, (2, 0, 1)),
            jnp.transpose(ops_out_t, (2, 0, 1)))


# speedup vs baseline: 8.1612x; 1.0600x over previous
import jax
import jax.numpy as jnp
from jax import lax
from jax.experimental import pallas as pl

_TI = 16


def _fused_kernel(adj_ref, uadj_ref, alpha_ref, uops_ref, adj_out_ref, ops_out_ref):
    ti = pl.program_id(0)
    tj = pl.program_id(1)
    nt = pl.num_programs(1)

    a = adj_ref[...]
    t = -jnp.log(uadj_ref[...])
    act = 1.0 / (1.0 + t * jnp.exp(-a))
    i = ti * _TI + lax.broadcasted_iota(jnp.int32, a.shape, 0)
    j = tj * _TI + lax.broadcasted_iota(jnp.int32, a.shape, 1)
    adj_out_ref[...] = jnp.where(j > i, act, 0.0)

    @pl.when(tj == nt - 1)
    def _ops():
        e = jnp.exp(alpha_ref[...]) / (-jnp.log(uops_ref[...]))
        ops_out_ref[...] = e / jnp.sum(e, axis=1, keepdims=True)


def kernel(adj_para, ops_alpha, u_adj, u_ops):
    B, N, _ = adj_para.shape
    OPS = ops_alpha.shape[-1]

    adj_t = jnp.transpose(adj_para, (1, 2, 0))
    uadj_t = jnp.transpose(u_adj, (1, 2, 0))
    alpha_t = jnp.transpose(ops_alpha, (1, 2, 0))
    uops_t = jnp.transpose(u_ops, (1, 2, 0))

    nt = N // _TI
    adj_in_spec = pl.BlockSpec((_TI, _TI, B), lambda ti, tj: (ti, jnp.maximum(tj, ti), 0))
    adj_out_spec = pl.BlockSpec((_TI, _TI, B), lambda ti, tj: (ti, tj, 0))
    ops_spec = pl.BlockSpec((_TI, OPS, B), lambda ti, tj: (ti, 0, 0))

    adj_out_t, ops_out_t = pl.pallas_call(
        _fused_kernel,
        grid=(nt, nt),
        in_specs=[adj_in_spec, adj_in_spec, ops_spec, ops_spec],
        out_specs=[adj_out_spec, ops_spec],
        out_shape=[
            jax.ShapeDtypeStruct((N, N, B), adj_para.dtype),
            jax.ShapeDtypeStruct((N, OPS, B), ops_alpha.dtype),
        ],
    )(adj_t, uadj_t, alpha_t, uops_t)

    return (jnp.transpose(adj_out_t, (2, 0, 1)),
            jnp.transpose(ops_out_t, (2, 0, 1)))
